# baseline (speedup 1.0000x reference)
import functools

import jax
import jax.numpy as jnp
from jax import lax
from jax.experimental import pallas as pl
from jax.experimental.pallas import tpu as pltpu
from jax.experimental.pallas import tpu_sc as plsc

IMAGE_SIZE = 128
K = 8
ZNEAR = 0.01
INF = float("inf")
NP = 5120
ROWS_PER_WORKER = IMAGE_SIZE // 8


def _transform(pts, M, eps=1e-6):
    ones = jnp.ones(pts.shape[:-1] + (1,), pts.dtype)
    p = jnp.concatenate([pts, ones], axis=-1)
    out = jnp.matmul(p.astype(jnp.bfloat16), M.astype(jnp.bfloat16),
                     preferred_element_type=jnp.float32)
    w = out[..., 3:4]
    w = jnp.where(jnp.abs(w) < eps, jnp.where(w >= 0, eps, -eps), w)
    return out[..., :3] / w


def _iota16():
    return lax.broadcasted_iota(jnp.int32, (16,), 0)


def _bcast(s, dtype=None):
    v = jnp.broadcast_to(s, (16,))
    return v.astype(dtype) if dtype is not None else v


def _sc_kernel(px_hbm, py_hbm, pz_hbm, r_hbm, idx_hbm, z_hbm, d_hbm,
               tpx, tpy, tpz, tr, cx0, cnx, cr0, cy1,
               cpx, cpy, cr2, ckey, flatp, cntp,
               sidx, sz, sd):
    wid = lax.axis_index("s") * 2 + lax.axis_index("c")
    b = wid >> 3
    res = wid & 7
    it = _iota16()
    itf = it.astype(jnp.float32)

    pltpu.sync_copy(px_hbm.at[b], tpx)
    pltpu.sync_copy(py_hbm.at[b], tpy)
    pltpu.sync_copy(pz_hbm.at[b], tpz)
    pltpu.sync_copy(r_hbm.at[b], tr)

    def filt(i, cnt):
        s = pl.ds(i * 16, 16)
        pxv = tpx[s]
        pyv = tpy[s]
        pzv = tpz[s]
        rv = tr[s]
        pxc = jnp.clip(pxv, -16.0, 16.0)
        pyc = jnp.clip(pyv, -16.0, 16.0)
        y0 = ((pyc - rv + 1.0) * 64.0 + 1023.5).astype(jnp.int32) - 1024
        y1 = ((pyc + rv + 1.0) * 64.0 + 1023.5).astype(jnp.int32) - 1024
        y0c = jnp.clip(y0, 0, 127)
        y1c = jnp.clip(y1, 0, 127)
        r0 = y0c + ((_bcast(res) - y0c) & 7)
        m = (r0 <= y1c) & (pzv > ZNEAR)
        pos = cnt + jnp.cumsum(m.astype(jnp.int32)) - 1
        x0 = ((pxc - rv + 1.0) * 64.0 + 1023.5).astype(jnp.int32) - 1024
        x1 = ((pxc + rv + 1.0) * 64.0 + 1023.5).astype(jnp.int32) - 1024
        x0c = jnp.clip(x0, 0, 127)
        nx = jnp.clip(x1, 0, 127) - x0c + 1
        key = lax.bitcast_convert_type(
            jnp.bitwise_or(
                jnp.bitwise_and(lax.bitcast_convert_type(pzv, jnp.int32),
                                jnp.int32(-8192)), i * 16 + it), jnp.float32)
        plsc.store_scatter(cx0, [pos], x0c, mask=m)
        plsc.store_scatter(cnx, [pos], nx, mask=m)
        plsc.store_scatter(cr0, [pos], r0, mask=m)
        plsc.store_scatter(cy1, [pos], y1c, mask=m)
        plsc.store_scatter(cpx, [pos], pxv, mask=m)
        plsc.store_scatter(cpy, [pos], pyv, mask=m)
        plsc.store_scatter(cr2, [pos], rv * rv, mask=m)
        plsc.store_scatter(ckey, [pos], key, mask=m)
        return cnt + jnp.sum(m.astype(jnp.int32))

    cnt = lax.fori_loop(0, NP // 16, filt, jnp.int32(0))
    tail = pl.ds(cnt, 16)
    cx0[tail] = jnp.zeros((16,), jnp.int32)
    cnx[tail] = jnp.ones((16,), jnp.int32)
    cr0[tail] = jnp.full((16,), 8, jnp.int32)
    cy1[tail] = jnp.zeros((16,), jnp.int32)

    NPIX = ROWS_PER_WORKER * IMAGE_SIZE

    def init(i, _):
        flatp[pl.ds(i * 16, 16)] = jnp.full((16,), INF, jnp.float32)
        return 0

    lax.fori_loop(0, NPIX * K // 16, init, 0)

    def initc(i, _):
        cntp[pl.ds(i * 16, 16)] = jnp.zeros((16,), jnp.int32)
        return 0

    lax.fori_loop(0, NPIX // 16, initc, 0)

    def insert_chunk(ci, _):
        s = pl.ds(ci * 16, 16)
        vx0 = cx0[s]
        vnx = cnx[s]
        vr0 = cr0[s]
        vy1 = cy1[s]
        vpx = cpx[s]
        vpy = cpy[s]
        vr2 = cr2[s]
        vkey = ckey[s]
        for t in range(16):
            x0 = vx0[t]
            r0 = vr0[t]
            y1 = vy1[t]
            pxs = _bcast(vpx[t])
            pys = _bcast(vpy[t])
            r2s = _bcast(vr2[t])
            keys = _bcast(vkey[t])
            lanemask = it < _bcast(vnx[t])
            addrx = jnp.minimum(_bcast(x0) + it, 127)
            xcv = (_bcast(x0).astype(jnp.float32) + itf + 0.5) * (1.0 / 64.0) - 1.0
            dxv = xcv - pxs
            dx2 = dxv * dxv
            trips = jnp.maximum(((y1 - r0) >> 3) + 1, 0)

            def row(rt, _, r0=r0, pys=pys, r2s=r2s, keys=keys,
                    lanemask=lanemask, addrx=addrx, dx2=dx2):
                ry = r0 + rt * 8
                base = (ry >> 3) * IMAGE_SIZE
                ycv = (_bcast(ry).astype(jnp.float32) * (1.0 / 64.0)
                       + (0.5 / 64.0 - 1.0))
                dyv = ycv - pys
                d2 = dx2 + dyv * dyv
                inserting = (d2 < r2s) & lanemask
                nz = plsc.all_reduce_population_count(inserting)[0]

                @pl.when(nz > 0)
                def _():
                    newkey = jnp.where(inserting, keys, INF)
                    addr = _bcast(base) + addrx
                    nk = newkey
                    for k in range(K):
                        ak = addr + _bcast(k * NPIX)
                        cur = plsc.load_gather(flatp, [ak])
                        plsc.store_scatter(flatp, [ak], jnp.minimum(cur, nk),
                                           mask=inserting)
                        nk = jnp.maximum(cur, nk)

                return 0

            lax.fori_loop(0, trips, row, 0)
        return 0

    lax.fori_loop(0, (cnt + 15) >> 4, insert_chunk, 0)

    def outrow(prow, _):
        ry = prow * 8 + res
        ycv = _bcast(ry).astype(jnp.float32) * (1.0 / 64.0) + (0.5 / 64.0 - 1.0)

        def chunk(c, _):
            xb = c * 16
            start = prow * IMAGE_SIZE + xb
            xv = _bcast(xb) + it
            xcv = xv.astype(jnp.float32) * (1.0 / 64.0) + (0.5 / 64.0 - 1.0)
            for k in range(K):
                keyv = flatp[pl.ds(start + k * NPIX, 16)]
                valid = keyv < INF
                kb = lax.bitcast_convert_type(keyv, jnp.int32)
                idxv = jnp.bitwise_and(kb, jnp.int32(8191))
                zv = lax.bitcast_convert_type(
                    jnp.bitwise_and(kb, jnp.int32(-8192)), jnp.float32)
                gpx = plsc.load_gather(tpx, [idxv])
                gpy = plsc.load_gather(tpy, [idxv])
                dxv = xcv - gpx
                dyv = ycv - gpy
                d2 = dxv * dxv + dyv * dyv
                kvec = jnp.full((16,), k, jnp.int32)
                plsc.store_scatter(sidx, [xv, kvec],
                                   jnp.where(valid, idxv, -1))
                plsc.store_scatter(sz, [xv, kvec],
                                   jnp.where(valid, zv, -1.0))
                plsc.store_scatter(sd, [xv, kvec],
                                   jnp.where(valid, d2, -1.0))
            return 0

        lax.fori_loop(0, IMAGE_SIZE // 16, chunk, 0)
        pltpu.sync_copy(sidx, idx_hbm.at[b, ry])
        pltpu.sync_copy(sz, z_hbm.at[b, ry])
        pltpu.sync_copy(sd, d_hbm.at[b, ry])
        return 0

    lax.fori_loop(0, ROWS_PER_WORKER, outrow, 0)


def kernel(points, radii, world_to_view, proj):
    B, N, _ = points.shape
    S = IMAGE_SIZE
    pts_view = _transform(points, world_to_view)
    pts_screen = _transform(pts_view, proj)
    px = pts_screen[..., 0]
    py = pts_screen[..., 1]
    pz = pts_view[..., 2]

    pad = NP - N
    px = jnp.pad(px, ((0, 0), (0, pad)))
    py = jnp.pad(py, ((0, 0), (0, pad)))
    pz = jnp.pad(pz, ((0, 0), (0, pad)))
    r = jnp.pad(radii, ((0, 0), (0, pad)))

    mesh = plsc.VectorSubcoreMesh(core_axis_name="c", subcore_axis_name="s")
    f = pl.kernel(
        _sc_kernel,
        out_type=[
            jax.ShapeDtypeStruct((B, S, S, K), jnp.int32),
            jax.ShapeDtypeStruct((B, S, S, K), jnp.float32),
            jax.ShapeDtypeStruct((B, S, S, K), jnp.float32),
        ],
        mesh=mesh,
        compiler_params=pltpu.CompilerParams(needs_layout_passes=False),
        scratch_types=(
            [pltpu.VMEM((NP,), jnp.float32)] * 4
            + [pltpu.VMEM((NP + 16,), jnp.int32)] * 4
            + [pltpu.VMEM((NP + 16,), jnp.float32)] * 4
            + [pltpu.VMEM((ROWS_PER_WORKER * S * K,), jnp.float32),
               pltpu.VMEM((ROWS_PER_WORKER * S,), jnp.int32)]
            + [pltpu.VMEM((S, K), jnp.int32),
               pltpu.VMEM((S, K), jnp.float32),
               pltpu.VMEM((S, K), jnp.float32)]
        ),
    )
    return tuple(f(px, py, pz, r))


# speedup vs baseline: 2.5892x; 2.5892x over previous
import jax
import jax.numpy as jnp
from jax import lax
from jax.experimental import pallas as pl
from jax.experimental.pallas import tpu as pltpu
from jax.experimental.pallas import tpu_sc as plsc

IMAGE_SIZE = 128
K = 8
ZNEAR = 0.01
INF = float("inf")
NP = 5120
ROWS_PER_WORKER = IMAGE_SIZE // 8
NPIX = ROWS_PER_WORKER * IMAGE_SIZE


def _transform(pts, M, eps=1e-6):
    ones = jnp.ones(pts.shape[:-1] + (1,), pts.dtype)
    p = jnp.concatenate([pts, ones], axis=-1)
    out = jnp.matmul(p.astype(jnp.bfloat16), M.astype(jnp.bfloat16),
                     preferred_element_type=jnp.float32)
    w = out[..., 3:4]
    w = jnp.where(jnp.abs(w) < eps, jnp.where(w >= 0, eps, -eps), w)
    return out[..., :3] / w


def _iota16():
    return lax.broadcasted_iota(jnp.int32, (16,), 0)


def _bcast(s, dtype=None):
    v = jnp.broadcast_to(s, (16,))
    return v.astype(dtype) if dtype is not None else v


def _sc_kernel(px_hbm, py_hbm, pz_hbm, r_hbm, idx_hbm, z_hbm, d_hbm,
               tpx, tpy, tpz, tr, cx0, cnx, cr0, cy1,
               cpx, cpy, cr2, ckey, p0, p1, p2, p3, p4, p5, p6, p7,
               sidx, sz, sd):
    planes = (p0, p1, p2, p3, p4, p5, p6, p7)
    wid = lax.axis_index("s") * 2 + lax.axis_index("c")
    b = wid >> 3
    res = wid & 7
    it = _iota16()
    itf = it.astype(jnp.float32)

    pltpu.sync_copy(px_hbm.at[b], tpx)
    pltpu.sync_copy(py_hbm.at[b], tpy)
    pltpu.sync_copy(pz_hbm.at[b], tpz)
    pltpu.sync_copy(r_hbm.at[b], tr)

    def filt(i, cnt):
        s = pl.ds(i * 16, 16)
        pxv = tpx[s]
        pyv = tpy[s]
        pzv = tpz[s]
        rv = tr[s]
        pxc = jnp.clip(pxv, -16.0, 16.0)
        pyc = jnp.clip(pyv, -16.0, 16.0)
        y0 = ((pyc - rv + 1.0) * 64.0 + 1023.5).astype(jnp.int32) - 1024
        y1 = ((pyc + rv + 1.0) * 64.0 + 1023.5).astype(jnp.int32) - 1024
        y0c = jnp.clip(y0, 0, 127)
        y1c = jnp.clip(y1, 0, 127)
        r0 = y0c + ((_bcast(res) - y0c) & 7)
        m = (r0 <= y1c) & (pzv > ZNEAR)
        pos = cnt + jnp.cumsum(m.astype(jnp.int32)) - 1
        x0 = ((pxc - rv + 1.0) * 64.0 + 1023.5).astype(jnp.int32) - 1024
        x1 = ((pxc + rv + 1.0) * 64.0 + 1023.5).astype(jnp.int32) - 1024
        x0c = jnp.clip(x0, 0, 127)
        nx = jnp.clip(x1, 0, 127) - x0c + 1
        key = lax.bitcast_convert_type(
            jnp.bitwise_or(
                jnp.bitwise_and(lax.bitcast_convert_type(pzv, jnp.int32),
                                jnp.int32(-8192)), i * 16 + it), jnp.float32)
        plsc.store_scatter(cx0, [pos], x0c, mask=m)
        plsc.store_scatter(cnx, [pos], nx, mask=m)
        plsc.store_scatter(cr0, [pos], r0, mask=m)
        plsc.store_scatter(cy1, [pos], y1c, mask=m)
        plsc.store_scatter(cpx, [pos], pxv, mask=m)
        plsc.store_scatter(cpy, [pos], pyv, mask=m)
        plsc.store_scatter(cr2, [pos], rv * rv, mask=m)
        plsc.store_scatter(ckey, [pos], key, mask=m)
        return cnt + jnp.sum(m.astype(jnp.int32))

    cnt = lax.fori_loop(0, NP // 16, filt, jnp.int32(0))

    def init(i, _):
        s = pl.ds(i * 16, 16)
        full = jnp.full((16,), INF, jnp.float32)
        for p in planes:
            p[s] = full
        return 0

    lax.fori_loop(0, NPIX // 16, init, 0)

    def insert(i, _):
        s = pl.ds(i, 16)
        x0 = cx0[s][0]
        nx = cnx[s][0]
        r0 = cr0[s][0]
        y1 = cy1[s][0]
        pxs = _bcast(cpx[s][0])
        pys = _bcast(cpy[s][0])
        r2s = _bcast(cr2[s][0])
        keys = _bcast(ckey[s][0])
        lanemask = it < _bcast(nx)
        addrx = jnp.minimum(_bcast(x0) + it, 127)
        xcv = (_bcast(x0).astype(jnp.float32) + itf + 0.5) * (1.0 / 64.0) - 1.0
        dxv = xcv - pxs
        dx2 = dxv * dxv
        trips = jnp.maximum(((y1 - r0) >> 3) + 1, 0)

        def row(t, _):
            ry = r0 + t * 8
            base = (ry >> 3) * IMAGE_SIZE
            ycv = _bcast(ry).astype(jnp.float32) * (1.0 / 64.0) + (0.5 / 64.0 - 1.0)
            dyv = ycv - pys
            d2 = dx2 + dyv * dyv
            inserting = (d2 < r2s) & lanemask
            newkey = jnp.where(inserting, keys, INF)
            addr = _bcast(base) + addrx
            curs = [plsc.load_gather(p, [addr]) for p in planes]
            prev = None
            for k, p in enumerate(planes):
                nv = jnp.minimum(curs[k], newkey)
                if k:
                    nv = jnp.maximum(curs[k - 1], nv)
                plsc.store_scatter(p, [addr], nv, mask=inserting)
            return 0

        lax.fori_loop(0, trips, row, 0)
        return 0

    lax.fori_loop(0, cnt, insert, 0)

    def outrow(prow, _):
        ry = prow * 8 + res
        ycv = _bcast(ry).astype(jnp.float32) * (1.0 / 64.0) + (0.5 / 64.0 - 1.0)

        def chunk(c, _):
            xb = c * 16
            start = prow * IMAGE_SIZE + xb
            xv = _bcast(xb) + it
            xcv = xv.astype(jnp.float32) * (1.0 / 64.0) + (0.5 / 64.0 - 1.0)
            for k, p in enumerate(planes):
                keyv = p[pl.ds(start, 16)]
                valid = keyv < INF
                kb = lax.bitcast_convert_type(keyv, jnp.int32)
                idxv = jnp.bitwise_and(kb, jnp.int32(8191))
                zv = lax.bitcast_convert_type(
                    jnp.bitwise_and(kb, jnp.int32(-8192)), jnp.float32)
                gpx = plsc.load_gather(tpx, [idxv])
                gpy = plsc.load_gather(tpy, [idxv])
                dxv = xcv - gpx
                dyv = ycv - gpy
                d2 = dxv * dxv + dyv * dyv
                kvec = jnp.full((16,), k, jnp.int32)
                plsc.store_scatter(sidx, [xv, kvec],
                                   jnp.where(valid, idxv, -1))
                plsc.store_scatter(sz, [xv, kvec],
                                   jnp.where(valid, zv, -1.0))
                plsc.store_scatter(sd, [xv, kvec],
                                   jnp.where(valid, d2, -1.0))
            return 0

        lax.fori_loop(0, IMAGE_SIZE // 16, chunk, 0)
        pltpu.sync_copy(sidx, idx_hbm.at[b, ry])
        pltpu.sync_copy(sz, z_hbm.at[b, ry])
        pltpu.sync_copy(sd, d_hbm.at[b, ry])
        return 0

    lax.fori_loop(0, ROWS_PER_WORKER, outrow, 0)


def kernel(points, radii, world_to_view, proj):
    B, N, _ = points.shape
    S = IMAGE_SIZE
    pts_view = _transform(points, world_to_view)
    pts_screen = _transform(pts_view, proj)
    px = pts_screen[..., 0]
    py = pts_screen[..., 1]
    pz = pts_view[..., 2]

    pad = NP - N
    px = jnp.pad(px, ((0, 0), (0, pad)))
    py = jnp.pad(py, ((0, 0), (0, pad)))
    pz = jnp.pad(pz, ((0, 0), (0, pad)))
    r = jnp.pad(radii, ((0, 0), (0, pad)))

    mesh = plsc.VectorSubcoreMesh(core_axis_name="c", subcore_axis_name="s")
    f = pl.kernel(
        _sc_kernel,
        out_type=[
            jax.ShapeDtypeStruct((B, S, S, K), jnp.int32),
            jax.ShapeDtypeStruct((B, S, S, K), jnp.float32),
            jax.ShapeDtypeStruct((B, S, S, K), jnp.float32),
        ],
        mesh=mesh,
        compiler_params=pltpu.CompilerParams(needs_layout_passes=False),
        scratch_types=(
            [pltpu.VMEM((NP,), jnp.float32)] * 4
            + [pltpu.VMEM((NP + 16,), jnp.int32)] * 4
            + [pltpu.VMEM((NP + 16,), jnp.float32)] * 4
            + [pltpu.VMEM((NPIX,), jnp.float32)] * K
            + [pltpu.VMEM((S, K), jnp.int32),
               pltpu.VMEM((S, K), jnp.float32),
               pltpu.VMEM((S, K), jnp.float32)]
        ),
    )
    return tuple(f(px, py, pz, r))


# speedup vs baseline: 2.6173x; 1.0109x over previous
import jax
import jax.numpy as jnp
from jax import lax
from jax.experimental import pallas as pl
from jax.experimental.pallas import tpu as pltpu
from jax.experimental.pallas import tpu_sc as plsc

IMAGE_SIZE = 128
K = 8
ZNEAR = 0.01
INF = float("inf")
NP = 5120
ROWS_PER_WORKER = IMAGE_SIZE // 8
NPIX = ROWS_PER_WORKER * IMAGE_SIZE


def _transform(pts, M, eps=1e-6):
    ones = jnp.ones(pts.shape[:-1] + (1,), pts.dtype)
    p = jnp.concatenate([pts, ones], axis=-1)
    out = jnp.matmul(p.astype(jnp.bfloat16), M.astype(jnp.bfloat16),
                     preferred_element_type=jnp.float32)
    w = out[..., 3:4]
    w = jnp.where(jnp.abs(w) < eps, jnp.where(w >= 0, eps, -eps), w)
    return out[..., :3] / w


def _iota16():
    return lax.broadcasted_iota(jnp.int32, (16,), 0)


def _bcast(s, dtype=None):
    v = jnp.broadcast_to(s, (16,))
    return v.astype(dtype) if dtype is not None else v


def _sc_kernel(px_hbm, py_hbm, pz_hbm, r_hbm, idx_hbm, z_hbm, d_hbm,
               tpx, tpy, tpz, tr, cx0, cnx, cr0, cy1,
               cpx, cpy, cr2, ckey, p0, p1, p2, p3, p4, p5, p6, p7,
               sidx, sz, sd, sem):
    planes = (p0, p1, p2, p3, p4, p5, p6, p7)
    wid = lax.axis_index("s") * 2 + lax.axis_index("c")
    b = wid >> 3
    res = wid & 7
    it = _iota16()
    itf = it.astype(jnp.float32)

    cins = [pltpu.async_copy(px_hbm.at[b], tpx, sem),
            pltpu.async_copy(py_hbm.at[b], tpy, sem),
            pltpu.async_copy(pz_hbm.at[b], tpz, sem),
            pltpu.async_copy(r_hbm.at[b], tr, sem)]
    for c in cins:
        c.wait()

    def filt(i, cnt):
        s = pl.ds(i * 16, 16)
        pxv = tpx[s]
        pyv = tpy[s]
        pzv = tpz[s]
        rv = tr[s]
        pxc = jnp.clip(pxv, -16.0, 16.0)
        pyc = jnp.clip(pyv, -16.0, 16.0)
        y0 = ((pyc - rv + 1.0) * 64.0 + 1023.5).astype(jnp.int32) - 1024
        y1 = ((pyc + rv + 1.0) * 64.0 + 1023.5).astype(jnp.int32) - 1024
        y0c = jnp.clip(y0, 0, 127)
        y1c = jnp.clip(y1, 0, 127)
        r0 = y0c + ((_bcast(res) - y0c) & 7)
        m = (r0 <= y1c) & (pzv > ZNEAR)
        pos = cnt + jnp.cumsum(m.astype(jnp.int32)) - 1
        x0 = ((pxc - rv + 1.0) * 64.0 + 1023.5).astype(jnp.int32) - 1024
        x1 = ((pxc + rv + 1.0) * 64.0 + 1023.5).astype(jnp.int32) - 1024
        x0c = jnp.clip(x0, 0, 127)
        nx = jnp.clip(x1, 0, 127) - x0c + 1
        key = lax.bitcast_convert_type(
            jnp.bitwise_or(
                jnp.bitwise_and(lax.bitcast_convert_type(pzv, jnp.int32),
                                jnp.int32(-8192)), i * 16 + it), jnp.float32)
        plsc.store_scatter(cx0, [pos], x0c, mask=m)
        plsc.store_scatter(cnx, [pos], nx, mask=m)
        plsc.store_scatter(cr0, [pos], r0, mask=m)
        plsc.store_scatter(cy1, [pos], y1c, mask=m)
        plsc.store_scatter(cpx, [pos], pxv, mask=m)
        plsc.store_scatter(cpy, [pos], pyv, mask=m)
        plsc.store_scatter(cr2, [pos], rv * rv, mask=m)
        plsc.store_scatter(ckey, [pos], key, mask=m)
        return cnt + jnp.sum(m.astype(jnp.int32))

    cnt = lax.fori_loop(0, NP // 16, filt, jnp.int32(0))

    def init(i, _):
        s = pl.ds(i * 16, 16)
        full = jnp.full((16,), INF, jnp.float32)
        for p in planes:
            p[s] = full
        return 0

    lax.fori_loop(0, NPIX // 16, init, 0)

    def insert(i, _):
        s = pl.ds(i, 16)
        x0 = cx0[s][0]
        nx = cnx[s][0]
        r0 = cr0[s][0]
        y1 = cy1[s][0]
        pxs = _bcast(cpx[s][0])
        pys = _bcast(cpy[s][0])
        r2s = _bcast(cr2[s][0])
        keys = _bcast(ckey[s][0])
        lanemask = it < _bcast(nx)
        addrx = jnp.minimum(_bcast(x0) + it, 127)
        xcv = (_bcast(x0).astype(jnp.float32) + itf + 0.5) * (1.0 / 64.0) - 1.0
        dxv = xcv - pxs
        dx2 = dxv * dxv
        trips = jnp.maximum(((y1 - r0) >> 3) + 1, 0)

        def row(t, _):
            ry = r0 + t * 8
            base = (ry >> 3) * IMAGE_SIZE
            ycv = _bcast(ry).astype(jnp.float32) * (1.0 / 64.0) + (0.5 / 64.0 - 1.0)
            dyv = ycv - pys
            d2 = dx2 + dyv * dyv
            inserting = (d2 < r2s) & lanemask
            newkey = jnp.where(inserting, keys, INF)
            addr = _bcast(base) + addrx
            curs = [plsc.load_gather(p, [addr]) for p in planes]
            prev = None
            for k, p in enumerate(planes):
                nv = jnp.minimum(curs[k], newkey)
                if k:
                    nv = jnp.maximum(curs[k - 1], nv)
                plsc.store_scatter(p, [addr], nv, mask=inserting)
            return 0

        lax.fori_loop(0, trips, row, 0)
        return 0

    lax.fori_loop(0, cnt, insert, 0)

    def outrow(prow, _):
        ry = prow * 8 + res
        ycv = _bcast(ry).astype(jnp.float32) * (1.0 / 64.0) + (0.5 / 64.0 - 1.0)

        def chunk(c, _):
            xb = c * 16
            start = prow * IMAGE_SIZE + xb
            xv = _bcast(xb) + it
            xcv = xv.astype(jnp.float32) * (1.0 / 64.0) + (0.5 / 64.0 - 1.0)
            for k, p in enumerate(planes):
                keyv = p[pl.ds(start, 16)]
                valid = keyv < INF
                kb = lax.bitcast_convert_type(keyv, jnp.int32)
                idxv = jnp.bitwise_and(kb, jnp.int32(8191))
                zv = lax.bitcast_convert_type(
                    jnp.bitwise_and(kb, jnp.int32(-8192)), jnp.float32)
                gpx = plsc.load_gather(tpx, [idxv])
                gpy = plsc.load_gather(tpy, [idxv])
                dxv = xcv - gpx
                dyv = ycv - gpy
                d2 = dxv * dxv + dyv * dyv
                kvec = jnp.full((16,), k, jnp.int32)
                plsc.store_scatter(sidx, [xv, kvec],
                                   jnp.where(valid, idxv, -1))
                plsc.store_scatter(sz, [xv, kvec],
                                   jnp.where(valid, zv, -1.0))
                plsc.store_scatter(sd, [xv, kvec],
                                   jnp.where(valid, d2, -1.0))
            return 0

        lax.fori_loop(0, IMAGE_SIZE // 16, chunk, 0)
        couts = [pltpu.async_copy(sidx, idx_hbm.at[b, ry], sem),
                 pltpu.async_copy(sz, z_hbm.at[b, ry], sem),
                 pltpu.async_copy(sd, d_hbm.at[b, ry], sem)]
        for c in couts:
            c.wait()
        return 0

    lax.fori_loop(0, ROWS_PER_WORKER, outrow, 0)


def kernel(points, radii, world_to_view, proj):
    B, N, _ = points.shape
    S = IMAGE_SIZE
    pts_view = _transform(points, world_to_view)
    pts_screen = _transform(pts_view, proj)
    px = pts_screen[..., 0]
    py = pts_screen[..., 1]
    pz = pts_view[..., 2]

    pad = NP - N
    px = jnp.pad(px, ((0, 0), (0, pad)))
    py = jnp.pad(py, ((0, 0), (0, pad)))
    pz = jnp.pad(pz, ((0, 0), (0, pad)))
    r = jnp.pad(radii, ((0, 0), (0, pad)))

    mesh = plsc.VectorSubcoreMesh(core_axis_name="c", subcore_axis_name="s")
    f = pl.kernel(
        _sc_kernel,
        out_type=[
            jax.ShapeDtypeStruct((B, S, S, K), jnp.int32),
            jax.ShapeDtypeStruct((B, S, S, K), jnp.float32),
            jax.ShapeDtypeStruct((B, S, S, K), jnp.float32),
        ],
        mesh=mesh,
        compiler_params=pltpu.CompilerParams(needs_layout_passes=False),
        scratch_types=(
            [pltpu.VMEM((NP,), jnp.float32)] * 4
            + [pltpu.VMEM((NP + 16,), jnp.int32)] * 4
            + [pltpu.VMEM((NP + 16,), jnp.float32)] * 4
            + [pltpu.VMEM((NPIX,), jnp.float32)] * K
            + [pltpu.VMEM((S, K), jnp.int32),
               pltpu.VMEM((S, K), jnp.float32),
               pltpu.VMEM((S, K), jnp.float32),
               pltpu.SemaphoreType.DMA]
        ),
    )
    return tuple(f(px, py, pz, r))
